# baseline (speedup 1.0000x reference)
import jax
import jax.numpy as jnp
from jax import lax
from jax.experimental import pallas as pl
from jax.experimental.pallas import tpu as pltpu
from jax.experimental.pallas import tpu_sc as plsc

_R, _N = 128, 32768
_L = 16
_B = _N // 2
_C = _B // _L
_G = _C // _L
_T = _G // _L
_W = _B // 4
_NC, _NS = 2, 16
_NW = _NC * _NS
_RPW = _R // _NW
_NB = _RPW * 2


def _sc_body(x_hbm, m_hbm, o_hbm, xv, mv, ov, sums, sg, go, off, sems):
    wid = lax.axis_index("s") * _NC + lax.axis_index("c")
    lane = lax.iota(jnp.int32, _L)
    lane4 = lane // 4
    shifts = (lane % 4) * 8

    def start_in(b):
        p = b % 2
        row = wid * _RPW + b // 2
        h = b % 2
        return (
            pltpu.async_copy(
                x_hbm.at[row, pl.ds(h * _B, _B)], xv.at[p], sems.at[p]
            ),
            pltpu.async_copy(
                m_hbm.at[row, pl.ds(h * _W, _W)], mv.at[p], sems.at[2 + p]
            ),
        )

    def compute_block(p, base):
        pidx = jnp.full((_L,), p, jnp.int32)

        @plsc.parallel_loop(0, _C // 4, unroll=2)
        def _(i):
            for j in range(4):
                widx = i * _L + j * 4 + lane4
                wj = plsc.load_gather(mv, [pidx, widx])
                mf = ((wj >> shifts) & 1).astype(jnp.float32)
                o = (i * 4 + j) * _L
                ov[p, pl.ds(o, _L)] = jnp.cumsum(xv[p, pl.ds(o, _L)] * mf)

        @plsc.parallel_loop(0, _G, unroll=4)
        def _(g):
            idx = (g * _L + lane) * _L + (_L - 1)
            sums[pl.ds(g * _L, _L)] = plsc.load_gather(ov, [pidx, idx])

        @plsc.parallel_loop(0, _G, unroll=4)
        def _(g):
            sg[pl.ds(g * _L, _L)] = jnp.cumsum(sums[pl.ds(g * _L, _L)])

        def p3(t, carry):
            idx = (t * _L + lane) * _L + (_L - 1)
            gt = plsc.load_gather(sg, [idx])
            st = jnp.cumsum(gt)
            go[pl.ds(t * _L, _L)] = st - gt + carry
            return carry + jnp.sum(gt)
        total = lax.fori_loop(0, _T, p3, base)

        @plsc.parallel_loop(0, _T)
        def _(t):
            gov = go[pl.ds(t * _L, _L)]
            for j in range(_L):
                o = (t * _L + j) * _L
                off[pl.ds(o, _L)] = (
                    sg[pl.ds(o, _L)] - sums[pl.ds(o, _L)] + gov[j]
                )

        @plsc.parallel_loop(0, _G, unroll=2)
        def _(g):
            offv = off[pl.ds(g * _L, _L)]
            for j in range(_L):
                o = (g * _L + j) * _L
                ov[p, pl.ds(o, _L)] = ov[p, pl.ds(o, _L)] + offv[j]

        return total

    in_cps = {0: start_in(0), 1: start_in(1)}
    out_cps = {}
    base = jnp.float32(0.0)
    for b in range(_NB):
        p = b % 2
        if b >= 2:
            out_cps.pop(b - 2).wait()
        for cp in in_cps.pop(b):
            cp.wait()
        if b % 2 == 0:
            base = jnp.float32(0.0)
        base = compute_block(p, base)
        if b + 2 < _NB:
            in_cps[b + 2] = start_in(b + 2)
        row = wid * _RPW + b // 2
        out_cps[b] = pltpu.async_copy(
            ov.at[p], o_hbm.at[row, pl.ds((b % 2) * _B, _B)], sems.at[4 + p]
        )
    for cp in out_cps.values():
        cp.wait()


def kernel(x, mask):
    maski = lax.bitcast_convert_type(
        mask.astype(jnp.uint8).reshape(_R, _N // 4, 4), jnp.int32
    )
    f = pl.kernel(
        _sc_body,
        out_type=jax.ShapeDtypeStruct((_R, _N), jnp.float32),
        mesh=plsc.VectorSubcoreMesh(core_axis_name="c", subcore_axis_name="s"),
        scratch_types=[
            pltpu.VMEM((2, _B), jnp.float32),
            pltpu.VMEM((2, _W), jnp.int32),
            pltpu.VMEM((2, _B), jnp.float32),
            pltpu.VMEM((_C,), jnp.float32),
            pltpu.VMEM((_C,), jnp.float32),
            pltpu.VMEM((_G,), jnp.float32),
            pltpu.VMEM((_C,), jnp.float32),
            pltpu.SemaphoreType.DMA((6,)),
        ],
        compiler_params=pltpu.CompilerParams(needs_layout_passes=False),
    )
    return f(x, maski)


# speedup vs baseline: 1.0256x; 1.0256x over previous
import jax
import jax.numpy as jnp
from jax import lax
from jax.experimental import pallas as pl
from jax.experimental.pallas import tpu as pltpu
from jax.experimental.pallas import tpu_sc as plsc

_R, _N = 128, 32768
_L = 16
_B = _N // 2
_C = _B // _L
_G = _C // _L
_T = _G // _L
_W = _B // 4
_NC, _NS = 2, 16
_NW = _NC * _NS
_RPW = _R // _NW
_NB = _RPW * 2


def _vtake(v, idx):
    return lax.gather(
        v,
        idx[:, None],
        lax.GatherDimensionNumbers(
            offset_dims=(), collapsed_slice_dims=(0,), start_index_map=(0,)
        ),
        slice_sizes=(1,),
        mode=lax.GatherScatterMode.PROMISE_IN_BOUNDS,
    )


def _sc_body(x_hbm, m_hbm, o_hbm, xv, mv, ov, sums, sg, go, off, sems):
    wid = lax.axis_index("s") * _NC + lax.axis_index("c")
    lane = lax.iota(jnp.int32, _L)
    lane4 = lane // 4
    shifts = (lane % 4) * 8

    def start_in(b):
        p = b % 2
        row = wid * _RPW + b // 2
        h = b % 2
        return (
            pltpu.async_copy(
                x_hbm.at[row, pl.ds(h * _B, _B)], xv.at[p], sems.at[p]
            ),
            pltpu.async_copy(
                m_hbm.at[row, pl.ds(h * _W, _W)], mv.at[p], sems.at[2 + p]
            ),
        )

    def compute_block(p, base):
        pidx = jnp.full((_L,), p, jnp.int32)

        @plsc.parallel_loop(0, _C // 4, unroll=2)
        def _(i):
            w = mv[p, pl.ds(i * _L, _L)]
            for j in range(4):
                wj = _vtake(w, j * 4 + lane4)
                mf = ((wj >> shifts) & 1).astype(jnp.float32)
                o = (i * 4 + j) * _L
                ov[p, pl.ds(o, _L)] = jnp.cumsum(xv[p, pl.ds(o, _L)] * mf)

        @plsc.parallel_loop(0, _G, unroll=4)
        def _(g):
            idx = (g * _L + lane) * _L + (_L - 1)
            sums[pl.ds(g * _L, _L)] = plsc.load_gather(ov, [pidx, idx])

        @plsc.parallel_loop(0, _G, unroll=4)
        def _(g):
            sg[pl.ds(g * _L, _L)] = jnp.cumsum(sums[pl.ds(g * _L, _L)])

        def p3(t, carry):
            idx = (t * _L + lane) * _L + (_L - 1)
            gt = plsc.load_gather(sg, [idx])
            st = jnp.cumsum(gt)
            go[pl.ds(t * _L, _L)] = st - gt + carry
            return carry + jnp.sum(gt)
        total = lax.fori_loop(0, _T, p3, base)

        @plsc.parallel_loop(0, _T)
        def _(t):
            gov = go[pl.ds(t * _L, _L)]
            for j in range(_L):
                o = (t * _L + j) * _L
                off[pl.ds(o, _L)] = (
                    sg[pl.ds(o, _L)] - sums[pl.ds(o, _L)] + gov[j]
                )

        @plsc.parallel_loop(0, _G, unroll=4)
        def _(g):
            offv = off[pl.ds(g * _L, _L)]
            for j in range(_L):
                o = (g * _L + j) * _L
                ov[p, pl.ds(o, _L)] = ov[p, pl.ds(o, _L)] + offv[j]

        return total

    in_cps = {0: start_in(0), 1: start_in(1)}
    out_cps = {}
    base = jnp.float32(0.0)
    for b in range(_NB):
        p = b % 2
        if b >= 2:
            out_cps.pop(b - 2).wait()
        for cp in in_cps.pop(b):
            cp.wait()
        if b % 2 == 0:
            base = jnp.float32(0.0)
        base = compute_block(p, base)
        if b + 2 < _NB:
            in_cps[b + 2] = start_in(b + 2)
        row = wid * _RPW + b // 2
        out_cps[b] = pltpu.async_copy(
            ov.at[p], o_hbm.at[row, pl.ds((b % 2) * _B, _B)], sems.at[4 + p]
        )
    for cp in out_cps.values():
        cp.wait()


def kernel(x, mask):
    maski = lax.bitcast_convert_type(
        mask.astype(jnp.uint8).reshape(_R, _N // 4, 4), jnp.int32
    )
    f = pl.kernel(
        _sc_body,
        out_type=jax.ShapeDtypeStruct((_R, _N), jnp.float32),
        mesh=plsc.VectorSubcoreMesh(core_axis_name="c", subcore_axis_name="s"),
        scratch_types=[
            pltpu.VMEM((2, _B), jnp.float32),
            pltpu.VMEM((2, _W), jnp.int32),
            pltpu.VMEM((2, _B), jnp.float32),
            pltpu.VMEM((_C,), jnp.float32),
            pltpu.VMEM((_C,), jnp.float32),
            pltpu.VMEM((_G,), jnp.float32),
            pltpu.VMEM((_C,), jnp.float32),
            pltpu.SemaphoreType.DMA((6,)),
        ],
        compiler_params=pltpu.CompilerParams(needs_layout_passes=False),
    )
    return f(x, maski)
